# baseline (speedup 1.0000x reference)
import functools

import jax
import jax.numpy as jnp
from jax import lax
from jax.experimental import pallas as pl
from jax.experimental.pallas import tpu as pltpu
from jax.experimental.pallas import tpu_sc as plsc

_N = 10000
_E = 320000
_DIN = 128
_H = 16
_COUT = 64
_NLAYERS = 4

_NC = 2
_NS = 16
_NW = _NC * _NS
_SCHUNK = 2560
_NSCH = 4
_NSS = 4
_SUB = _SCHUNK // _NSS
_EPW = _SCHUNK * _NSCH
_EPAD = _EPW * _NW
_ACC_N = 10112
_RPT = _ACC_N // _NS

@functools.cache
def _build_seg_sum_sc():
    mesh = plsc.VectorSubcoreMesh(core_axis_name="c", subcore_axis_name="s")

    @functools.partial(
        pl.kernel,
        out_type=jax.ShapeDtypeStruct((_NC, _ACC_N, _H), jnp.float32),
        mesh=mesh,
        scratch_types=[
            pltpu.VMEM((_NSCH * _NSS, _SUB), jnp.int32),
            pltpu.VMEM((_NSCH * _NSS, _SUB), jnp.int32),
            pltpu.VMEM((_SCHUNK, _H), jnp.float32),
            pltpu.VMEM((_SCHUNK, _H), jnp.float32),
            pltpu.VMEM_SHARED((_ACC_N, _H), jnp.float32),
            pltpu.SemaphoreType.DMA,
            pltpu.SemaphoreType.DMA,
        ],
        compiler_params=pltpu.CompilerParams(use_tc_tiling_on_sc=False),
    )
    def _seg_sum_sc(y_hbm, src_hbm, dst_hbm, zero_hbm, out_hbm,
                    src_v, dst_v, rows_a, rows_b, acc, gsem, ssem):
        c = lax.axis_index("c")
        s = lax.axis_index("s")
        wid = c * _NS + s
        pltpu.sync_copy(zero_hbm.at[pl.ds(s * _RPT, _RPT)],
                        acc.at[pl.ds(s * _RPT, _RPT)])
        pltpu.sync_copy(src_hbm.at[wid], src_v)
        pltpu.sync_copy(dst_hbm.at[wid], dst_v)
        plsc.subcore_barrier()

        bufs = [rows_a, rows_b]

        def start_gathers(j):
            b = bufs[j % 2]
            return [pltpu.async_copy(y_hbm.at[src_v.at[j * _NSS + k]],
                                     b.at[pl.ds(k * _SUB, _SUB)], gsem)
                    for k in range(_NSS)]

        def start_scatters(j):
            b = bufs[j % 2]
            return [pltpu.async_copy(b.at[pl.ds(k * _SUB, _SUB)],
                                     acc.at[dst_v.at[j * _NSS + k]], ssem,
                                     add=True)
                    for k in range(_NSS)]

        gathers = [None] * _NSCH
        scatters = [None] * _NSCH
        gathers[0] = start_gathers(0)
        for j in range(_NSCH):
            for cp in gathers[j]:
                cp.wait()
            scatters[j] = start_scatters(j)
            if j + 1 < _NSCH:
                if j - 1 >= 0:
                    for cp in scatters[j - 1]:
                        cp.wait()
                gathers[j + 1] = start_gathers(j + 1)
        if _NSCH >= 2:
            for cp in scatters[_NSCH - 2]:
                cp.wait()
        for cp in scatters[_NSCH - 1]:
            cp.wait()
        plsc.subcore_barrier()
        pltpu.sync_copy(acc.at[pl.ds(s * _RPT, _RPT)],
                        out_hbm.at[c, pl.ds(s * _RPT, _RPT)])

    return _seg_sum_sc


def _mm0_body(x_ref, w_ref, y_ref):
    y_ref[...] = jnp.dot(x_ref[...], w_ref[...],
                         preferred_element_type=jnp.float32)


def _bn_relu(t, g, b):
    m = jnp.mean(t, axis=0, keepdims=True)
    v = jnp.mean((t - m) ** 2, axis=0, keepdims=True)
    return jnp.maximum((t - m) * lax.rsqrt(v + 1e-5) * g + b, 0.0)


def _layer_h(y_ref, seg_ref, w2_ref, g1_ref, b1_ref, g2_ref, b2_ref):
    t = y_ref[...] + seg_ref[0, :_N, :] + seg_ref[1, :_N, :]
    t = _bn_relu(t, g1_ref[...], b1_ref[...])
    t = jnp.dot(t, w2_ref[...], preferred_element_type=jnp.float32)
    return _bn_relu(t, g2_ref[...], b2_ref[...])


def _dense_body(y_ref, seg_ref, w2_ref, g1_ref, b1_ref, g2_ref, b2_ref,
                w1n_ref, yn_ref, cs_ref):
    h = _layer_h(y_ref, seg_ref, w2_ref, g1_ref, b1_ref, g2_ref, b2_ref)
    yn_ref[...] = jnp.dot(h, w1n_ref[...], preferred_element_type=jnp.float32)
    cs_ref[...] = jnp.sum(h, axis=0, keepdims=True)


def _final_body(y_ref, seg_ref, w2_ref, g1_ref, b1_ref, g2_ref, b2_ref,
                cs1_ref, cs2_ref, cs3_ref, pw_ref, pb_ref, score_ref):
    h = _layer_h(y_ref, seg_ref, w2_ref, g1_ref, b1_ref, g2_ref, b2_ref)
    cs4 = jnp.sum(h, axis=0, keepdims=True)
    score = pb_ref[...]
    for i, cs in enumerate((cs1_ref[...], cs2_ref[...], cs3_ref[...], cs4)):
        score = score + jnp.dot(cs, pw_ref[i * _H:(i + 1) * _H, :],
                                preferred_element_type=jnp.float32)
    score_ref[...] = score


def kernel(x, edge_index, params):
    f32 = jnp.float32
    src = edge_index[0]
    dst = edge_index[1]
    pad = _EPAD - _E
    src3 = jnp.concatenate(
        [src, jnp.zeros((pad,), jnp.int32)]).reshape(_NW, _NSCH * _NSS, _SUB)
    dst3 = jnp.concatenate(
        [dst, jnp.full((pad,), _N, jnp.int32)]).reshape(_NW, _NSCH * _NSS, _SUB)
    zeros = jnp.zeros((_ACC_N, _H), f32)

    y = pl.pallas_call(
        _mm0_body,
        out_shape=jax.ShapeDtypeStruct((_N, _H), f32),
    )(x, params['W1_0'])

    seg_sum_sc = _build_seg_sum_sc()
    cs = []
    for l in range(_NLAYERS):
        seg = seg_sum_sc(y, src3, dst3, zeros)
        g1 = params['bn1_g_%d' % l].reshape(1, _H)
        b1 = params['bn1_b_%d' % l].reshape(1, _H)
        g2 = params['bn2_g_%d' % l].reshape(1, _H)
        b2 = params['bn2_b_%d' % l].reshape(1, _H)
        w2 = params['W2_%d' % l]
        if l < _NLAYERS - 1:
            y, c = pl.pallas_call(
                _dense_body,
                out_shape=(jax.ShapeDtypeStruct((_N, _H), f32),
                           jax.ShapeDtypeStruct((1, _H), f32)),
            )(y, seg, w2, g1, b1, g2, b2, params['W1_%d' % (l + 1)])
            cs.append(c)
        else:
            pw_all = jnp.concatenate(
                [params['PW_%d' % i] for i in range(1, _NLAYERS + 1)], axis=0)
            pb_sum = (params['Pb_1'] + params['Pb_2'] + params['Pb_3']
                      + params['Pb_4']).reshape(1, _COUT)
            score = pl.pallas_call(
                _final_body,
                out_shape=jax.ShapeDtypeStruct((1, _COUT), f32),
            )(y, seg, w2, g1, b1, g2, b2, cs[0], cs[1], cs[2], pw_all, pb_sum)
    return score


# speedup vs baseline: 1.0272x; 1.0272x over previous
import functools

import jax
import jax.numpy as jnp
from jax import lax
from jax.experimental import pallas as pl
from jax.experimental.pallas import tpu as pltpu
from jax.experimental.pallas import tpu_sc as plsc

_N = 10000
_E = 320000
_DIN = 128
_H = 16
_COUT = 64
_NLAYERS = 4

_NC = 2
_NS = 16
_NW = _NC * _NS
_SCHUNK = 1280
_NSCH = 8
_NBUF = 4
_EPW = _SCHUNK * _NSCH
_EPAD = _EPW * _NW
_ACC_N = 10112
_RPT = _ACC_N // _NS

@functools.cache
def _build_seg_sum_sc():
    mesh = plsc.VectorSubcoreMesh(core_axis_name="c", subcore_axis_name="s")

    @functools.partial(
        pl.kernel,
        out_type=jax.ShapeDtypeStruct((_NC, _ACC_N, _H), jnp.float32),
        mesh=mesh,
        scratch_types=[
            pltpu.VMEM((_NSCH, _SCHUNK), jnp.int32),
            pltpu.VMEM((_NSCH, _SCHUNK), jnp.int32),
            pltpu.VMEM((_NBUF, _SCHUNK, _H), jnp.float32),
            pltpu.VMEM_SHARED((_ACC_N, _H), jnp.float32),
            pltpu.SemaphoreType.DMA,
            pltpu.SemaphoreType.DMA,
        ],
        compiler_params=pltpu.CompilerParams(use_tc_tiling_on_sc=False),
    )
    def _seg_sum_sc(y_hbm, src_hbm, dst_hbm, zero_hbm, out_hbm,
                    src_v, dst_v, rows_v, acc, gsem, ssem):
        c = lax.axis_index("c")
        s = lax.axis_index("s")
        wid = c * _NS + s
        pltpu.sync_copy(zero_hbm.at[pl.ds(s * _RPT, _RPT)],
                        acc.at[pl.ds(s * _RPT, _RPT)])
        pltpu.sync_copy(src_hbm.at[wid], src_v)
        pltpu.sync_copy(dst_hbm.at[wid], dst_v)
        plsc.subcore_barrier()

        def start_gather(j):
            return pltpu.async_copy(y_hbm.at[src_v.at[j]],
                                    rows_v.at[j % _NBUF], gsem)

        def start_scatter(j):
            return pltpu.async_copy(rows_v.at[j % _NBUF],
                                    acc.at[dst_v.at[j]], ssem, add=True)

        gathers = [None] * _NSCH
        scatters = [None] * _NSCH
        for j in range(_NBUF - 1):
            gathers[j] = start_gather(j)
        for j in range(_NSCH):
            gathers[j].wait()
            scatters[j] = start_scatter(j)
            nxt = j + _NBUF - 1
            if nxt < _NSCH:
                if j - 1 >= 0:
                    scatters[j - 1].wait()
                gathers[nxt] = start_gather(nxt)
        for j in range(_NSCH):
            if scatters[j] is not None and j >= _NSCH - _NBUF:
                scatters[j].wait()
        plsc.subcore_barrier()
        pltpu.sync_copy(acc.at[pl.ds(s * _RPT, _RPT)],
                        out_hbm.at[c, pl.ds(s * _RPT, _RPT)])

    return _seg_sum_sc


def _mm0_body(x_ref, w_ref, y_ref):
    y_ref[...] = jnp.dot(x_ref[...], w_ref[...],
                         preferred_element_type=jnp.float32)


def _bn_relu(t, g, b):
    m = jnp.mean(t, axis=0, keepdims=True)
    v = jnp.mean((t - m) ** 2, axis=0, keepdims=True)
    return jnp.maximum((t - m) * lax.rsqrt(v + 1e-5) * g + b, 0.0)


def _layer_h(y_ref, seg_ref, w2_ref, g1_ref, b1_ref, g2_ref, b2_ref):
    t = y_ref[...] + seg_ref[0, :_N, :] + seg_ref[1, :_N, :]
    t = _bn_relu(t, g1_ref[...], b1_ref[...])
    t = jnp.dot(t, w2_ref[...], preferred_element_type=jnp.float32)
    return _bn_relu(t, g2_ref[...], b2_ref[...])


def _dense_body(y_ref, seg_ref, w2_ref, g1_ref, b1_ref, g2_ref, b2_ref,
                w1n_ref, yn_ref, cs_ref):
    h = _layer_h(y_ref, seg_ref, w2_ref, g1_ref, b1_ref, g2_ref, b2_ref)
    yn_ref[...] = jnp.dot(h, w1n_ref[...], preferred_element_type=jnp.float32)
    cs_ref[...] = jnp.sum(h, axis=0, keepdims=True)


def _final_body(y_ref, seg_ref, w2_ref, g1_ref, b1_ref, g2_ref, b2_ref,
                cs1_ref, cs2_ref, cs3_ref, pw_ref, pb_ref, score_ref):
    h = _layer_h(y_ref, seg_ref, w2_ref, g1_ref, b1_ref, g2_ref, b2_ref)
    cs4 = jnp.sum(h, axis=0, keepdims=True)
    score = pb_ref[...]
    for i, cs in enumerate((cs1_ref[...], cs2_ref[...], cs3_ref[...], cs4)):
        score = score + jnp.dot(cs, pw_ref[i * _H:(i + 1) * _H, :],
                                preferred_element_type=jnp.float32)
    score_ref[...] = score


def kernel(x, edge_index, params):
    f32 = jnp.float32
    src = edge_index[0]
    dst = edge_index[1]
    pad = _EPAD - _E
    src3 = jnp.concatenate(
        [src, jnp.zeros((pad,), jnp.int32)]).reshape(_NW, _NSCH, _SCHUNK)
    dst3 = jnp.concatenate(
        [dst, jnp.full((pad,), _N, jnp.int32)]).reshape(_NW, _NSCH, _SCHUNK)
    zeros = jnp.zeros((_ACC_N, _H), f32)

    y = pl.pallas_call(
        _mm0_body,
        out_shape=jax.ShapeDtypeStruct((_N, _H), f32),
    )(x, params['W1_0'])

    seg_sum_sc = _build_seg_sum_sc()
    cs = []
    for l in range(_NLAYERS):
        seg = seg_sum_sc(y, src3, dst3, zeros)
        g1 = params['bn1_g_%d' % l].reshape(1, _H)
        b1 = params['bn1_b_%d' % l].reshape(1, _H)
        g2 = params['bn2_g_%d' % l].reshape(1, _H)
        b2 = params['bn2_b_%d' % l].reshape(1, _H)
        w2 = params['W2_%d' % l]
        if l < _NLAYERS - 1:
            y, c = pl.pallas_call(
                _dense_body,
                out_shape=(jax.ShapeDtypeStruct((_N, _H), f32),
                           jax.ShapeDtypeStruct((1, _H), f32)),
            )(y, seg, w2, g1, b1, g2, b2, params['W1_%d' % (l + 1)])
            cs.append(c)
        else:
            pw_all = jnp.concatenate(
                [params['PW_%d' % i] for i in range(1, _NLAYERS + 1)], axis=0)
            pb_sum = (params['Pb_1'] + params['Pb_2'] + params['Pb_3']
                      + params['Pb_4']).reshape(1, _COUT)
            score = pl.pallas_call(
                _final_body,
                out_shape=jax.ShapeDtypeStruct((1, _COUT), f32),
            )(y, seg, w2, g1, b1, g2, b2, cs[0], cs[1], cs[2], pw_all, pb_sum)
    return score


# speedup vs baseline: 1.0338x; 1.0064x over previous
import functools

import jax
import jax.numpy as jnp
from jax import lax
from jax.experimental import pallas as pl
from jax.experimental.pallas import tpu as pltpu
from jax.experimental.pallas import tpu_sc as plsc

_N = 10000
_E = 320000
_DIN = 128
_H = 16
_COUT = 64
_NLAYERS = 4

_NC = 2
_NS = 16
_NW = _NC * _NS
_SCHUNK = 1280
_G0 = 6
_G1 = 10
_NCHT = _NS * (_G0 + _G1)
_NBUF = 4
_EPAD = _SCHUNK * _NCHT
_ACC_N = 10112
_RPT = _ACC_N // _NS

@functools.cache
def _build_seg_sum_sc():
    mesh = plsc.VectorSubcoreMesh(core_axis_name="c", subcore_axis_name="s")

    @functools.partial(
        pl.kernel,
        out_type=jax.ShapeDtypeStruct((_NC, _ACC_N, _H), jnp.float32),
        mesh=mesh,
        scratch_types=[
            pltpu.VMEM((max(_G0, _G1), _SCHUNK), jnp.int32),
            pltpu.VMEM((max(_G0, _G1), _SCHUNK), jnp.int32),
            pltpu.VMEM((_NBUF, _SCHUNK, _H), jnp.float32),
            pltpu.VMEM_SHARED((_ACC_N, _H), jnp.float32),
            pltpu.SemaphoreType.DMA,
            pltpu.SemaphoreType.DMA,
        ],
        compiler_params=pltpu.CompilerParams(use_tc_tiling_on_sc=False),
    )
    def _seg_sum_sc(y_hbm, src_hbm, dst_hbm, zero_hbm, out_hbm,
                    src_v, dst_v, rows_v, acc, gsem, ssem):
        c = lax.axis_index("c")
        s = lax.axis_index("s")
        pltpu.sync_copy(zero_hbm.at[pl.ds(s * _RPT, _RPT)],
                        acc.at[pl.ds(s * _RPT, _RPT)])

        def pipeline(base, ng):
            pltpu.sync_copy(src_hbm.at[pl.ds(base, ng)], src_v.at[pl.ds(0, ng)])
            pltpu.sync_copy(dst_hbm.at[pl.ds(base, ng)], dst_v.at[pl.ds(0, ng)])
            plsc.subcore_barrier()

            def start_gather(j):
                return pltpu.async_copy(y_hbm.at[src_v.at[j]],
                                        rows_v.at[j % _NBUF], gsem)

            def start_scatter(j):
                return pltpu.async_copy(rows_v.at[j % _NBUF],
                                        acc.at[dst_v.at[j]], ssem, add=True)

            gathers = [None] * ng
            scatters = [None] * ng
            for j in range(min(_NBUF - 1, ng)):
                gathers[j] = start_gather(j)
            for j in range(ng):
                gathers[j].wait()
                scatters[j] = start_scatter(j)
                nxt = j + _NBUF - 1
                if nxt < ng:
                    if j - 1 >= 0:
                        scatters[j - 1].wait()
                    gathers[nxt] = start_gather(nxt)
            for j in range(ng):
                if scatters[j] is not None and j >= ng - _NBUF:
                    scatters[j].wait()

        @pl.when(c == 0)
        def _():
            pipeline(s * _G0, _G0)

        @pl.when(c == 1)
        def _():
            pipeline(_NS * _G0 + s * _G1, _G1)
        plsc.subcore_barrier()
        pltpu.sync_copy(acc.at[pl.ds(s * _RPT, _RPT)],
                        out_hbm.at[c, pl.ds(s * _RPT, _RPT)])

    return _seg_sum_sc


def _mm0_body(x_ref, w_ref, y_ref):
    y_ref[...] = jnp.dot(x_ref[...], w_ref[...],
                         preferred_element_type=jnp.float32)


def _bn_relu(t, g, b):
    m = jnp.mean(t, axis=0, keepdims=True)
    v = jnp.mean((t - m) ** 2, axis=0, keepdims=True)
    return jnp.maximum((t - m) * lax.rsqrt(v + 1e-5) * g + b, 0.0)


def _layer_h(y_ref, seg_ref, w2_ref, g1_ref, b1_ref, g2_ref, b2_ref):
    t = y_ref[...] + seg_ref[0, :_N, :] + seg_ref[1, :_N, :]
    t = _bn_relu(t, g1_ref[...], b1_ref[...])
    t = jnp.dot(t, w2_ref[...], preferred_element_type=jnp.float32)
    return _bn_relu(t, g2_ref[...], b2_ref[...])


def _dense_body(y_ref, seg_ref, w2_ref, g1_ref, b1_ref, g2_ref, b2_ref,
                w1n_ref, yn_ref, cs_ref):
    h = _layer_h(y_ref, seg_ref, w2_ref, g1_ref, b1_ref, g2_ref, b2_ref)
    yn_ref[...] = jnp.dot(h, w1n_ref[...], preferred_element_type=jnp.float32)
    cs_ref[...] = jnp.sum(h, axis=0, keepdims=True)


def _final_body(y_ref, seg_ref, w2_ref, g1_ref, b1_ref, g2_ref, b2_ref,
                cs1_ref, cs2_ref, cs3_ref, pw_ref, pb_ref, score_ref):
    h = _layer_h(y_ref, seg_ref, w2_ref, g1_ref, b1_ref, g2_ref, b2_ref)
    cs4 = jnp.sum(h, axis=0, keepdims=True)
    score = pb_ref[...]
    for i, cs in enumerate((cs1_ref[...], cs2_ref[...], cs3_ref[...], cs4)):
        score = score + jnp.dot(cs, pw_ref[i * _H:(i + 1) * _H, :],
                                preferred_element_type=jnp.float32)
    score_ref[...] = score


def kernel(x, edge_index, params):
    f32 = jnp.float32
    src = edge_index[0]
    dst = edge_index[1]
    pad = _EPAD - _E
    src3 = jnp.concatenate(
        [src, jnp.zeros((pad,), jnp.int32)]).reshape(_NCHT, _SCHUNK)
    dst3 = jnp.concatenate(
        [dst, jnp.full((pad,), _N, jnp.int32)]).reshape(_NCHT, _SCHUNK)
    zeros = jnp.zeros((_ACC_N, _H), f32)

    y = pl.pallas_call(
        _mm0_body,
        out_shape=jax.ShapeDtypeStruct((_N, _H), f32),
    )(x, params['W1_0'])

    seg_sum_sc = _build_seg_sum_sc()
    cs = []
    for l in range(_NLAYERS):
        seg = seg_sum_sc(y, src3, dst3, zeros)
        g1 = params['bn1_g_%d' % l].reshape(1, _H)
        b1 = params['bn1_b_%d' % l].reshape(1, _H)
        g2 = params['bn2_g_%d' % l].reshape(1, _H)
        b2 = params['bn2_b_%d' % l].reshape(1, _H)
        w2 = params['W2_%d' % l]
        if l < _NLAYERS - 1:
            y, c = pl.pallas_call(
                _dense_body,
                out_shape=(jax.ShapeDtypeStruct((_N, _H), f32),
                           jax.ShapeDtypeStruct((1, _H), f32)),
            )(y, seg, w2, g1, b1, g2, b2, params['W1_%d' % (l + 1)])
            cs.append(c)
        else:
            pw_all = jnp.concatenate(
                [params['PW_%d' % i] for i in range(1, _NLAYERS + 1)], axis=0)
            pb_sum = (params['Pb_1'] + params['Pb_2'] + params['Pb_3']
                      + params['Pb_4']).reshape(1, _COUT)
            score = pl.pallas_call(
                _final_body,
                out_shape=jax.ShapeDtypeStruct((1, _COUT), f32),
            )(y, seg, w2, g1, b1, g2, b2, cs[0], cs[1], cs[2], pw_all, pb_sum)
    return score


# speedup vs baseline: 1.0529x; 1.0185x over previous
import functools

import jax
import jax.numpy as jnp
from jax import lax
from jax.experimental import pallas as pl
from jax.experimental.pallas import tpu as pltpu
from jax.experimental.pallas import tpu_sc as plsc

_N = 10000
_E = 320000
_DIN = 128
_H = 16
_COUT = 64
_NLAYERS = 4

_NC = 2
_NS = 16
_NW = _NC * _NS
_SCHUNK = 1280
_G0 = 10
_G1 = 6
_NCHT = _NS * (_G0 + _G1)
_NBUF = 4
_EPAD = _SCHUNK * _NCHT
_ACC_N = 10112
_RPT = _ACC_N // _NS

@functools.cache
def _build_seg_sum_sc():
    mesh = plsc.VectorSubcoreMesh(core_axis_name="c", subcore_axis_name="s")

    @functools.partial(
        pl.kernel,
        out_type=jax.ShapeDtypeStruct((_NC, _ACC_N, _H), jnp.float32),
        mesh=mesh,
        scratch_types=[
            pltpu.VMEM((max(_G0, _G1), _SCHUNK), jnp.int32),
            pltpu.VMEM((max(_G0, _G1), _SCHUNK), jnp.int32),
            pltpu.VMEM((_NBUF, _SCHUNK, _H), jnp.float32),
            pltpu.VMEM_SHARED((_ACC_N, _H), jnp.float32),
            pltpu.SemaphoreType.DMA,
            pltpu.SemaphoreType.DMA,
        ],
        compiler_params=pltpu.CompilerParams(use_tc_tiling_on_sc=False),
    )
    def _seg_sum_sc(y_hbm, src_hbm, dst_hbm, zero_hbm, out_hbm,
                    src_v, dst_v, rows_v, acc, gsem, ssem):
        c = lax.axis_index("c")
        s = lax.axis_index("s")
        pltpu.sync_copy(zero_hbm.at[pl.ds(s * _RPT, _RPT)],
                        acc.at[pl.ds(s * _RPT, _RPT)])

        def pipeline(base, ng):
            pltpu.sync_copy(src_hbm.at[pl.ds(base, ng)], src_v.at[pl.ds(0, ng)])
            pltpu.sync_copy(dst_hbm.at[pl.ds(base, ng)], dst_v.at[pl.ds(0, ng)])
            plsc.subcore_barrier()

            def start_gather(j):
                return pltpu.async_copy(y_hbm.at[src_v.at[j]],
                                        rows_v.at[j % _NBUF], gsem)

            def start_scatter(j):
                return pltpu.async_copy(rows_v.at[j % _NBUF],
                                        acc.at[dst_v.at[j]], ssem, add=True)

            gathers = [None] * ng
            scatters = [None] * ng
            for j in range(min(_NBUF - 1, ng)):
                gathers[j] = start_gather(j)
            for j in range(ng):
                gathers[j].wait()
                scatters[j] = start_scatter(j)
                nxt = j + _NBUF - 1
                if nxt < ng:
                    if j - 1 >= 0:
                        scatters[j - 1].wait()
                    gathers[nxt] = start_gather(nxt)
            for j in range(ng):
                if scatters[j] is not None and j >= ng - _NBUF:
                    scatters[j].wait()

        @pl.when(c == 0)
        def _():
            pipeline(s * _G0, _G0)

        @pl.when(c == 1)
        def _():
            pipeline(_NS * _G0 + s * _G1, _G1)
        plsc.subcore_barrier()
        pltpu.sync_copy(acc.at[pl.ds(s * _RPT, _RPT)],
                        out_hbm.at[c, pl.ds(s * _RPT, _RPT)])

    return _seg_sum_sc


def _mm0_body(x_ref, w_ref, y_ref):
    y_ref[...] = jnp.dot(x_ref[...], w_ref[...],
                         preferred_element_type=jnp.float32)


def _bn_relu(t, g, b):
    m = jnp.mean(t, axis=0, keepdims=True)
    v = jnp.mean((t - m) ** 2, axis=0, keepdims=True)
    return jnp.maximum((t - m) * lax.rsqrt(v + 1e-5) * g + b, 0.0)


def _layer_h(y_ref, seg_ref, w2_ref, g1_ref, b1_ref, g2_ref, b2_ref):
    t = y_ref[...] + seg_ref[0, :_N, :] + seg_ref[1, :_N, :]
    t = _bn_relu(t, g1_ref[...], b1_ref[...])
    t = jnp.dot(t, w2_ref[...], preferred_element_type=jnp.float32)
    return _bn_relu(t, g2_ref[...], b2_ref[...])


def _dense_body(y_ref, seg_ref, w2_ref, g1_ref, b1_ref, g2_ref, b2_ref,
                w1n_ref, yn_ref, cs_ref):
    h = _layer_h(y_ref, seg_ref, w2_ref, g1_ref, b1_ref, g2_ref, b2_ref)
    yn_ref[...] = jnp.dot(h, w1n_ref[...], preferred_element_type=jnp.float32)
    cs_ref[...] = jnp.sum(h, axis=0, keepdims=True)


def _final_body(y_ref, seg_ref, w2_ref, g1_ref, b1_ref, g2_ref, b2_ref,
                cs1_ref, cs2_ref, cs3_ref, pw_ref, pb_ref, score_ref):
    h = _layer_h(y_ref, seg_ref, w2_ref, g1_ref, b1_ref, g2_ref, b2_ref)
    cs4 = jnp.sum(h, axis=0, keepdims=True)
    score = pb_ref[...]
    for i, cs in enumerate((cs1_ref[...], cs2_ref[...], cs3_ref[...], cs4)):
        score = score + jnp.dot(cs, pw_ref[i * _H:(i + 1) * _H, :],
                                preferred_element_type=jnp.float32)
    score_ref[...] = score


def kernel(x, edge_index, params):
    f32 = jnp.float32
    src = edge_index[0]
    dst = edge_index[1]
    pad = _EPAD - _E
    src3 = jnp.concatenate(
        [src, jnp.zeros((pad,), jnp.int32)]).reshape(_NCHT, _SCHUNK)
    dst3 = jnp.concatenate(
        [dst, jnp.full((pad,), _N, jnp.int32)]).reshape(_NCHT, _SCHUNK)
    zeros = jnp.zeros((_ACC_N, _H), f32)

    y = pl.pallas_call(
        _mm0_body,
        out_shape=jax.ShapeDtypeStruct((_N, _H), f32),
    )(x, params['W1_0'])

    seg_sum_sc = _build_seg_sum_sc()
    cs = []
    for l in range(_NLAYERS):
        seg = seg_sum_sc(y, src3, dst3, zeros)
        g1 = params['bn1_g_%d' % l].reshape(1, _H)
        b1 = params['bn1_b_%d' % l].reshape(1, _H)
        g2 = params['bn2_g_%d' % l].reshape(1, _H)
        b2 = params['bn2_b_%d' % l].reshape(1, _H)
        w2 = params['W2_%d' % l]
        if l < _NLAYERS - 1:
            y, c = pl.pallas_call(
                _dense_body,
                out_shape=(jax.ShapeDtypeStruct((_N, _H), f32),
                           jax.ShapeDtypeStruct((1, _H), f32)),
            )(y, seg, w2, g1, b1, g2, b2, params['W1_%d' % (l + 1)])
            cs.append(c)
        else:
            pw_all = jnp.concatenate(
                [params['PW_%d' % i] for i in range(1, _NLAYERS + 1)], axis=0)
            pb_sum = (params['Pb_1'] + params['Pb_2'] + params['Pb_3']
                      + params['Pb_4']).reshape(1, _COUT)
            score = pl.pallas_call(
                _final_body,
                out_shape=jax.ShapeDtypeStruct((1, _COUT), f32),
            )(y, seg, w2, g1, b1, g2, b2, cs[0], cs[1], cs[2], pw_all, pb_sum)
    return score
